# baseline (speedup 1.0000x reference)
import functools

import jax
import jax.numpy as jnp
from jax import lax
from jax.experimental import pallas as pl
from jax.experimental.pallas import tpu as pltpu
from jax.experimental.pallas import tpu_sc as plsc

B_TOTAL = 16384
D = 128

_SECTIONS = (
    ("species",   1025, 1,    0, 2824, True),
    ("moves",      920, 2, 1032, 2832, False),
    ("abilities",  310, 1, 1952, 2840, True),
    ("items",      560, 1, 2264, 2848, True),
)
TABLE_ROWS = 2856


def _build_table_body(*refs):
    (s_unk, s_dat, m_unk, m_dat, a_unk, a_dat, i_unk, i_dat,
     sw1, sb1, sw2, sb2, sw3, sb3,
     mw1, mb1, mw2, mb2, mw3, mb3,
     aw1, ab1, aw2, ab2, aw3, ab3,
     iw1, ib1, iw2, ib2, iw3, ib3, out_ref) = refs
    comp = {
        "species": (s_unk, s_dat, sw1, sb1, sw2, sb2, sw3, sb3),
        "moves": (m_unk, m_dat, mw1, mb1, mw2, mb2, mw3, mb3),
        "abilities": (a_unk, a_dat, aw1, ab1, aw2, ab2, aw3, ab3),
        "items": (i_unk, i_dat, iw1, ib1, iw2, ib2, iw3, ib3),
    }

    def _norm(o):
        n = jnp.sqrt(jnp.sum(o * o, axis=1, keepdims=True))
        return o / jnp.maximum(n, 1e-12)

    dn = (((1,), (1,)), ((), ()))
    for name, n_rows, nu, off, uoff, norm in _SECTIONS:
        unk, dat, w1, b1, w2, b2, w3, b3 = comp[name]
        x = dat[...]
        h = jax.nn.relu(
            lax.dot_general(x, w1[...], dn, preferred_element_type=jnp.float32) + b1[...])
        h = jax.nn.relu(
            lax.dot_general(h, w2[...], dn, preferred_element_type=jnp.float32) + b2[...])
        h = lax.dot_general(h, w3[...], dn, preferred_element_type=jnp.float32) + b3[...]
        u = unk[...]
        if norm:
            h = _norm(h)
            u = _norm(u)
        out_ref[off:off + n_rows, :] = h
        out_ref[uoff:uoff + nu, :] = u


def _build_table(unk_dat, weight_list):
    return pl.pallas_call(
        _build_table_body,
        out_shape=jax.ShapeDtypeStruct((TABLE_ROWS, D), jnp.float32),
    )(*unk_dat, *weight_list)


_CHUNK = 32
_RPT = 7


def _make_gather_kernel():
    mesh = plsc.VectorSubcoreMesh(core_axis_name="c", subcore_axis_name="s")
    nw = mesh.num_cores * mesh.num_subcores
    per_w = B_TOTAL // nw
    n_chunks = per_w // _CHUNK

    def body(table, s_i, m0_i, m1_i, m2_i, m3_i, a_i, i_i, out,
             iv0, iv1, iv2, iv3, iv4, iv5, iv6, rows0, rows1, ab0, mb0,
             ab1, mb1, nbuf, nbuf2, sbuf, gsem0, gsem1, ssem0, ssem1):
        sid = lax.axis_index("s")
        wid = sid * mesh.num_cores + lax.axis_index("c")
        wbase = wid * per_w
        idxv = (iv0, iv1, iv2, iv3, iv4, iv5, iv6)
        for j, src in enumerate((s_i, m0_i, m1_i, m2_i, m3_i, a_i, i_i)):
            pltpu.sync_copy(src.at[pl.ds(wbase, per_w)], idxv[j])
        rows = (rows0, rows1)
        ab = (ab0, ab1)
        mb = (mb0, mb1)
        gsem = (gsem0, gsem1)
        ssem = (ssem0, ssem1)
        lane16 = lax.iota(jnp.int32, 16)

        mbuf = mb0
        del mb1

        def fire_gather(g, buf):
            sl = pl.ds(g * _CHUNK, _CHUNK)
            for j in range(_RPT):
                pltpu.async_copy(table.at[idxv[j].at[sl]],
                                 rows[buf].at[j * _CHUNK:(j + 1) * _CHUNK],
                                 gsem[buf])

        def wait_gather(buf):
            for j in range(_RPT):
                pltpu.make_async_copy(
                    table.at[idxv[j].at[pl.ds(0, _CHUNK)]],
                    rows[buf].at[j * _CHUNK:(j + 1) * _CHUNK],
                    gsem[buf]).wait()

        def wait_store(buf):
            pltpu.make_async_copy(ab[buf], out.at[pl.ds(0, _CHUNK)],
                                  ssem[buf]).wait()

        fire_gather(0, 0)
        fire_gather(1, 1)

        def pair(p, carry):
            for cur in range(2):
                g = p * 2 + cur
                wait_gather(cur)

                @pl.when(g >= 2)
                def _():
                    wait_store(cur)
                rbuf = rows[cur]
                abuf = ab[cur]

                _chunk_compute(g, rbuf, abuf, mbuf, nbuf, nbuf2, sbuf, lane16)
                base = wbase + g * _CHUNK
                pltpu.async_copy(abuf, out.at[pl.ds(base, _CHUNK)], ssem[cur])

                @pl.when(g + 2 < n_chunks)
                def _():
                    fire_gather(g + 2, cur)
            return carry

        lax.fori_loop(0, n_chunks // 2, pair, 0)
        wait_store(0)
        wait_store(1)

    def _chunk_compute(g, rbuf, abuf, mbuf, nbuf, nbuf2, sbuf, lane16):
            @plsc.parallel_loop(0, _CHUNK, unroll=8)
            def accum(t):
                svec = None
                for k in range(D // 16):
                    sl = pl.ds(k * 16, 16)
                    m = ((rbuf[_CHUNK + t, sl] + rbuf[2 * _CHUNK + t, sl])
                         + (rbuf[3 * _CHUNK + t, sl] + rbuf[4 * _CHUNK + t, sl]))
                    mbuf[t, sl] = m
                    abuf[t, sl] = ((rbuf[t, sl] + rbuf[5 * _CHUNK + t, sl])
                                   + rbuf[6 * _CHUNK + t, sl])
                    sq = m * m
                    svec = sq if svec is None else svec + sq
                t16 = t * 16
                nbuf[pl.ds(t16, 16)] = svec
                nbuf2[pl.ds(t16, 16)] = nbuf[pl.ds(t16, 16)] + nbuf[pl.ds(t16 + 8, 16)]
                nbuf[pl.ds(t16, 16)] = nbuf2[pl.ds(t16, 16)] + nbuf2[pl.ds(t16 + 4, 16)]
                nbuf2[pl.ds(t16, 16)] = nbuf[pl.ds(t16, 16)] + nbuf[pl.ds(t16 + 2, 16)]
                nbuf[pl.ds(t16, 16)] = nbuf2[pl.ds(t16, 16)] + nbuf2[pl.ds(t16 + 1, 16)]


            for h in range(_CHUNK // 16):
                hs = pl.ds(h * 16, 16)
                ss = None
                for l in range(16):
                    v = nbuf[pl.ds((h * 16 + l) * 16, 16)]
                    tot = lax.broadcast_in_dim(v[0], (16,), ())
                    ss = tot if ss is None else jnp.where(lane16 == l, tot, ss)
                seed = jnp.int32(0x5F3759DF) - (
                    lax.bitcast_convert_type(ss, jnp.int32) >> 1)
                y = lax.bitcast_convert_type(seed, jnp.float32)
                hh = ss * 0.5
                for _ in range(3):
                    y = y * (1.5 - hh * y * y)
                sbuf[hs] = jnp.minimum(y, 1e12)

            @plsc.parallel_loop(0, _CHUNK, unroll=4)
            def scale_out(t):
                sc = lax.broadcast_in_dim(sbuf[pl.ds(t, 16)][0], (16,), ())
                for k in range(D // 16):
                    sl = pl.ds(k * 16, 16)
                    abuf[t, sl] = abuf[t, sl] + mbuf[t, sl] * sc

    return pl.kernel(
        body,
        out_type=jax.ShapeDtypeStruct((B_TOTAL, D), jnp.float32),
        mesh=mesh,
        scratch_types=(
            pltpu.VMEM((per_w,), jnp.int32),
            pltpu.VMEM((per_w,), jnp.int32),
            pltpu.VMEM((per_w,), jnp.int32),
            pltpu.VMEM((per_w,), jnp.int32),
            pltpu.VMEM((per_w,), jnp.int32),
            pltpu.VMEM((per_w,), jnp.int32),
            pltpu.VMEM((per_w,), jnp.int32),
            pltpu.VMEM((_RPT * _CHUNK, D), jnp.float32),
            pltpu.VMEM((_RPT * _CHUNK, D), jnp.float32),
            pltpu.VMEM((_CHUNK, D), jnp.float32),
            pltpu.VMEM((_CHUNK, D), jnp.float32),
            pltpu.VMEM((_CHUNK, D), jnp.float32),
            pltpu.VMEM((_CHUNK, D), jnp.float32),
            pltpu.VMEM((16 * _CHUNK + 16,), jnp.float32),
            pltpu.VMEM((16 * _CHUNK + 16,), jnp.float32),
            pltpu.VMEM((_CHUNK + 16,), jnp.float32),
            pltpu.SemaphoreType.DMA,
            pltpu.SemaphoreType.DMA,
            pltpu.SemaphoreType.DMA,
            pltpu.SemaphoreType.DMA,
        ),
    )


def kernel(species_indices, moves_indices, abilities_indices, items_indices, species_unknown, species_data, species_W1, species_b1, species_W2, species_b2, species_W3, species_b3, moves_unknown, moves_data, moves_W1, moves_b1, moves_W2, moves_b2, moves_W3, moves_b3, abilities_unknown, abilities_data, abilities_W1, abilities_b1, abilities_W2, abilities_b2, abilities_W3, abilities_b3, items_unknown, items_data, items_W1, items_b1, items_W2, items_b2, items_W3, items_b3):
    f32 = jnp.float32
    unk_dat = [a.astype(f32) for a in (
        species_unknown, species_data, moves_unknown, moves_data,
        abilities_unknown, abilities_data, items_unknown, items_data)]
    weight_list = [w.astype(f32) for w in (
        species_W1, species_b1, species_W2, species_b2, species_W3, species_b3,
        moves_W1, moves_b1, moves_W2, moves_b2, moves_W3, moves_b3,
        abilities_W1, abilities_b1, abilities_W2, abilities_b2, abilities_W3, abilities_b3,
        items_W1, items_b1, items_W2, items_b2, items_W3, items_b3)]
    table = _build_table(unk_dat, weight_list)

    i32 = jnp.int32

    def row_idx(idx, n, nu, off, uoff):
        idx = idx.astype(i32)
        return jnp.where(idx < nu,
                         uoff + jnp.clip(idx, 0, nu - 1),
                         off + jnp.clip(idx - nu, 0, n - 1)).astype(i32)

    s_i = row_idx(species_indices, 1025, 1, 0, 2824)
    a_i = row_idx(abilities_indices, 310, 1, 1952, 2840)
    i_i = row_idx(items_indices, 560, 1, 2264, 2848)
    m = row_idx(moves_indices, 920, 2, 1032, 2832)

    gather = _make_gather_kernel()
    return gather(table, s_i, m[:, 0], m[:, 1], m[:, 2], m[:, 3], a_i, i_i)


# speedup vs baseline: 1.0895x; 1.0895x over previous
import jax
import jax.numpy as jnp
from jax import lax
from jax.experimental import pallas as pl
from jax.experimental.pallas import tpu as pltpu
from jax.experimental.pallas import tpu_sc as plsc

B_TOTAL = 16384
D = 128

_SECTIONS = (
    ("species",   1025, 1,    0, 2824, True),
    ("moves",      920, 2, 1032, 2832, False),
    ("abilities",  310, 1, 1952, 2840, True),
    ("items",      560, 1, 2264, 2848, True),
)
TABLE_ROWS = 2856


def _build_table_body(*refs):
    (s_unk, s_dat, m_unk, m_dat, a_unk, a_dat, i_unk, i_dat,
     sw1, sb1, sw2, sb2, sw3, sb3,
     mw1, mb1, mw2, mb2, mw3, mb3,
     aw1, ab1, aw2, ab2, aw3, ab3,
     iw1, ib1, iw2, ib2, iw3, ib3, out_ref) = refs
    comp = {
        "species": (s_unk, s_dat, sw1, sb1, sw2, sb2, sw3, sb3),
        "moves": (m_unk, m_dat, mw1, mb1, mw2, mb2, mw3, mb3),
        "abilities": (a_unk, a_dat, aw1, ab1, aw2, ab2, aw3, ab3),
        "items": (i_unk, i_dat, iw1, ib1, iw2, ib2, iw3, ib3),
    }

    def _norm(o):
        n = jnp.sqrt(jnp.sum(o * o, axis=1, keepdims=True))
        return o / jnp.maximum(n, 1e-12)

    dn = (((1,), (1,)), ((), ()))
    for name, n_rows, nu, off, uoff, norm in _SECTIONS:
        unk, dat, w1, b1, w2, b2, w3, b3 = comp[name]
        x = dat[...]
        h = jax.nn.relu(
            lax.dot_general(x, w1[...], dn, preferred_element_type=jnp.float32) + b1[...])
        h = jax.nn.relu(
            lax.dot_general(h, w2[...], dn, preferred_element_type=jnp.float32) + b2[...])
        h = lax.dot_general(h, w3[...], dn, preferred_element_type=jnp.float32) + b3[...]
        u = unk[...]
        if norm:
            h = _norm(h)
            u = _norm(u)
        out_ref[off:off + n_rows, :] = h
        out_ref[uoff:uoff + nu, :] = u


def _build_table(unk_dat, weight_list):
    return pl.pallas_call(
        _build_table_body,
        out_shape=jax.ShapeDtypeStruct((TABLE_ROWS, D), jnp.float32),
    )(*unk_dat, *weight_list)


_CHUNK = 32
_RPT = 7


def _make_gather_kernel():
    mesh = plsc.VectorSubcoreMesh(core_axis_name="c", subcore_axis_name="s")
    nw = mesh.num_cores * mesh.num_subcores
    per_w = B_TOTAL // nw
    n_chunks = per_w // _CHUNK

    def body(table, s_i, m0_i, m1_i, m2_i, m3_i, a_i, i_i, out,
             iv0, iv1, iv2, iv3, iv4, iv5, iv6, rows0, rows1, ab0, mb0,
             ab1, mb1, nbuf, nbuf2, sbuf, gsem0, gsem1, ssem0, ssem1):
        sid = lax.axis_index("s")
        wid = sid * mesh.num_cores + lax.axis_index("c")
        wbase = wid * per_w
        idxv = (iv0, iv1, iv2, iv3, iv4, iv5, iv6)
        for j, src in enumerate((s_i, m0_i, m1_i, m2_i, m3_i, a_i, i_i)):
            pltpu.sync_copy(src.at[pl.ds(wbase, per_w)], idxv[j])
        rows = (rows0, rows1)
        ab = (ab0, ab1)
        mb = (mb0, mb1)
        gsem = (gsem0, gsem1)
        ssem = (ssem0, ssem1)
        lane16 = lax.iota(jnp.int32, 16)

        mbuf = mb0
        del mb1

        def fire_gather(g, buf):
            sl = pl.ds(g * _CHUNK, _CHUNK)
            for j in range(_RPT):
                pltpu.async_copy(table.at[idxv[j].at[sl]],
                                 rows[buf].at[j * _CHUNK:(j + 1) * _CHUNK],
                                 gsem[buf])

        def wait_gather(buf):
            for j in range(_RPT):
                pltpu.make_async_copy(
                    table.at[idxv[j].at[pl.ds(0, _CHUNK)]],
                    rows[buf].at[j * _CHUNK:(j + 1) * _CHUNK],
                    gsem[buf]).wait()

        def wait_store(buf):
            pltpu.make_async_copy(ab[buf], out.at[pl.ds(0, _CHUNK)],
                                  ssem[buf]).wait()

        fire_gather(0, 0)
        fire_gather(1, 1)

        def pair(p, carry):
            for cur in range(2):
                g = p * 2 + cur
                wait_gather(cur)

                @pl.when(g >= 2)
                def _():
                    wait_store(cur)
                rbuf = rows[cur]
                abuf = ab[cur]

                _chunk_compute(g, rbuf, abuf, mbuf, nbuf, nbuf2, sbuf, lane16)
                base = wbase + g * _CHUNK
                pltpu.async_copy(abuf, out.at[pl.ds(base, _CHUNK)], ssem[cur])

                @pl.when(g + 2 < n_chunks)
                def _():
                    fire_gather(g + 2, cur)
            return carry

        lax.fori_loop(0, n_chunks // 2, pair, 0)
        wait_store(0)
        wait_store(1)

    def _chunk_compute(g, rbuf, abuf, mbuf, nbuf, nbuf2, sbuf, lane16):
            @plsc.parallel_loop(0, _CHUNK, unroll=4)
            def accum(t):
                svec = None
                for k in range(D // 16):
                    sl = pl.ds(k * 16, 16)
                    m = ((rbuf[_CHUNK + t, sl] + rbuf[2 * _CHUNK + t, sl])
                         + (rbuf[3 * _CHUNK + t, sl] + rbuf[4 * _CHUNK + t, sl]))
                    mbuf[t, sl] = m
                    abuf[t, sl] = ((rbuf[t, sl] + rbuf[5 * _CHUNK + t, sl])
                                   + rbuf[6 * _CHUNK + t, sl])
                    sq = m * m
                    svec = sq if svec is None else svec + sq
                t16 = t * 16
                nbuf[pl.ds(t16, 16)] = svec
                nbuf2[pl.ds(t16, 16)] = nbuf[pl.ds(t16, 16)] + nbuf[pl.ds(t16 + 8, 16)]
                nbuf[pl.ds(t16, 16)] = nbuf2[pl.ds(t16, 16)] + nbuf2[pl.ds(t16 + 4, 16)]
                nbuf2[pl.ds(t16, 16)] = nbuf[pl.ds(t16, 16)] + nbuf[pl.ds(t16 + 2, 16)]
                nbuf[pl.ds(t16, 16)] = nbuf2[pl.ds(t16, 16)] + nbuf2[pl.ds(t16 + 1, 16)]


            for h in range(_CHUNK // 16):
                hs = pl.ds(h * 16, 16)
                ss = None
                for l in range(16):
                    v = nbuf[pl.ds((h * 16 + l) * 16, 16)]
                    tot = lax.broadcast_in_dim(v[0], (16,), ())
                    ss = tot if ss is None else jnp.where(lane16 == l, tot, ss)
                seed = jnp.int32(0x5F3759DF) - (
                    lax.bitcast_convert_type(ss, jnp.int32) >> 1)
                y = lax.bitcast_convert_type(seed, jnp.float32)
                hh = ss * 0.5
                for _ in range(3):
                    y = y * (1.5 - hh * y * y)
                sbuf[hs] = jnp.minimum(y, 1e12)

            @plsc.parallel_loop(0, _CHUNK, unroll=4)
            def scale_out(t):
                sc = lax.broadcast_in_dim(sbuf[pl.ds(t, 16)][0], (16,), ())
                for k in range(D // 16):
                    sl = pl.ds(k * 16, 16)
                    abuf[t, sl] = abuf[t, sl] + mbuf[t, sl] * sc

    return pl.kernel(
        body,
        out_type=jax.ShapeDtypeStruct((B_TOTAL, D), jnp.float32),
        mesh=mesh,
        scratch_types=(
            pltpu.VMEM((per_w,), jnp.int32),
            pltpu.VMEM((per_w,), jnp.int32),
            pltpu.VMEM((per_w,), jnp.int32),
            pltpu.VMEM((per_w,), jnp.int32),
            pltpu.VMEM((per_w,), jnp.int32),
            pltpu.VMEM((per_w,), jnp.int32),
            pltpu.VMEM((per_w,), jnp.int32),
            pltpu.VMEM((_RPT * _CHUNK, D), jnp.float32),
            pltpu.VMEM((_RPT * _CHUNK, D), jnp.float32),
            pltpu.VMEM((_CHUNK, D), jnp.float32),
            pltpu.VMEM((_CHUNK, D), jnp.float32),
            pltpu.VMEM((_CHUNK, D), jnp.float32),
            pltpu.VMEM((_CHUNK, D), jnp.float32),
            pltpu.VMEM((16 * _CHUNK + 16,), jnp.float32),
            pltpu.VMEM((16 * _CHUNK + 16,), jnp.float32),
            pltpu.VMEM((_CHUNK + 16,), jnp.float32),
            pltpu.SemaphoreType.DMA,
            pltpu.SemaphoreType.DMA,
            pltpu.SemaphoreType.DMA,
            pltpu.SemaphoreType.DMA,
        ),
    )


def kernel(species_indices, moves_indices, abilities_indices, items_indices, species_unknown, species_data, species_W1, species_b1, species_W2, species_b2, species_W3, species_b3, moves_unknown, moves_data, moves_W1, moves_b1, moves_W2, moves_b2, moves_W3, moves_b3, abilities_unknown, abilities_data, abilities_W1, abilities_b1, abilities_W2, abilities_b2, abilities_W3, abilities_b3, items_unknown, items_data, items_W1, items_b1, items_W2, items_b2, items_W3, items_b3):
    f32 = jnp.float32
    unk_dat = [a.astype(f32) for a in (
        species_unknown, species_data, moves_unknown, moves_data,
        abilities_unknown, abilities_data, items_unknown, items_data)]
    weight_list = [w.astype(f32) for w in (
        species_W1, species_b1, species_W2, species_b2, species_W3, species_b3,
        moves_W1, moves_b1, moves_W2, moves_b2, moves_W3, moves_b3,
        abilities_W1, abilities_b1, abilities_W2, abilities_b2, abilities_W3, abilities_b3,
        items_W1, items_b1, items_W2, items_b2, items_W3, items_b3)]
    table = _build_table(unk_dat, weight_list)

    i32 = jnp.int32

    def row_idx(idx, n, nu, off, uoff):
        idx = idx.astype(i32)
        return jnp.where(idx < nu,
                         uoff + jnp.clip(idx, 0, nu - 1),
                         off + jnp.clip(idx - nu, 0, n - 1)).astype(i32)

    s_i = row_idx(species_indices, 1025, 1, 0, 2824)
    a_i = row_idx(abilities_indices, 310, 1, 1952, 2840)
    i_i = row_idx(items_indices, 560, 1, 2264, 2848)
    m = row_idx(moves_indices, 920, 2, 1032, 2832)

    gather = _make_gather_kernel()
    return gather(table, s_i, m[:, 0], m[:, 1], m[:, 2], m[:, 3], a_i, i_i)
